# baseline (speedup 1.0000x reference)
import functools

import jax
import jax.numpy as jnp
from jax import lax
from jax.experimental import pallas as pl
from jax.experimental.pallas import tpu as pltpu
from jax.experimental.pallas import tpu_sc as plsc

_N = 10000
_E = 320000
_D = 128
_K = 4
_STEPS = 5

_NC = 2
_NS = 16
_C = 64
_CH = 320
_NBUF = 5
_NQ = 4
_HCH = _CH // _NQ
_EPW = _C * _CH
_EPAD = _NS * _EPW - _E
_NPAD = 10240
_ZR = _NPAD // _NS

_BN = 2000


def _transform2_body(x_ref, w0_ref, b0_ref, w1_ref, b1_ref, out_ref):
  xv = x_ref[...]
  for l, (w_ref, b_ref) in enumerate(((w0_ref, b0_ref), (w1_ref, b1_ref))):
    for k in range(_K):
      out_ref[l, k] = (jnp.dot(xv, w_ref[k], preferred_element_type=jnp.float32)
                       + b_ref[k][None, :])


_w_spec = pl.BlockSpec((_K, _D, _D), lambda i: (0, 0, 0))
_b_spec = pl.BlockSpec((_K, _D), lambda i: (0, 0))


def _transform2(x, w0, b0, w1, b1):
  return pl.pallas_call(
      _transform2_body,
      grid=(_N // _BN,),
      in_specs=[pl.BlockSpec((_BN, _D), lambda i: (i, 0)),
                _w_spec, _b_spec, _w_spec, _b_spec],
      out_specs=pl.BlockSpec((_NC, _K, _BN, _D), lambda i: (0, 0, i, 0)),
      out_shape=jax.ShapeDtypeStruct((_NC, _K, _N, _D), jnp.float32),
  )(x, w0, b0, w1, b1)


def _gru_math(a, h, wih_t, whh_t, bih, bhh):
  gi = jnp.dot(a, wih_t, preferred_element_type=jnp.float32) + bih
  gh = jnp.dot(h, whh_t, preferred_element_type=jnp.float32) + bhh
  r = jax.nn.sigmoid(gi[:, :_D] + gh[:, :_D])
  z = jax.nn.sigmoid(gi[:, _D:2 * _D] + gh[:, _D:2 * _D])
  n = jnp.tanh(gi[:, 2 * _D:] + r * gh[:, 2 * _D:])
  return (1.0 - z) * n + z * h


def _gru2_body(a0_ref, a1_ref, h0_ref, h1_ref,
               wih0_ref, whh0_ref, bih0_ref, bhh0_ref,
               wih1_ref, whh1_ref, bih1_ref, bhh1_ref,
               h0o_ref, h1o_ref):
  h0o_ref[...] = _gru_math(a0_ref[...], h0_ref[...], wih0_ref[...],
                           whh0_ref[...], bih0_ref[...], bhh0_ref[...])
  h1o_ref[...] = _gru_math(a1_ref[...], h1_ref[...], wih1_ref[...],
                           whh1_ref[...], bih1_ref[...], bhh1_ref[...])


def _gru2_tr_body(a0_ref, a1_ref, h0_ref, h1_ref,
                  wih0_ref, whh0_ref, bih0_ref, bhh0_ref,
                  wih1_ref, whh1_ref, bih1_ref, bhh1_ref,
                  w0_ref, b0_ref, w1_ref, b1_ref,
                  h0o_ref, h1o_ref, wh_ref):
  for l, (a_ref, h_ref, wih_ref, whh_ref, bih_ref, bhh_ref, w_ref, b_ref,
          ho_ref) in enumerate((
              (a0_ref, h0_ref, wih0_ref, whh0_ref, bih0_ref, bhh0_ref,
               w0_ref, b0_ref, h0o_ref),
              (a1_ref, h1_ref, wih1_ref, whh1_ref, bih1_ref, bhh1_ref,
               w1_ref, b1_ref, h1o_ref))):
    hn = _gru_math(a_ref[...], h_ref[...], wih_ref[...], whh_ref[...],
                   bih_ref[...], bhh_ref[...])
    ho_ref[...] = hn
    for k in range(_K):
      wh_ref[l, k] = (jnp.dot(hn, w_ref[k], preferred_element_type=jnp.float32)
                      + b_ref[k][None, :])


_row_spec = pl.BlockSpec((_BN, _D), lambda i: (i, 0))
_w3_spec = pl.BlockSpec((_D, 3 * _D), lambda i: (0, 0))
_b3_spec = pl.BlockSpec((1, 3 * _D), lambda i: (0, 0))


def _gru2(a0, a1, h0, h1, g0, g1):
  return pl.pallas_call(
      _gru2_body,
      grid=(_N // _BN,),
      in_specs=[_row_spec] * 4 + [_w3_spec, _w3_spec, _b3_spec, _b3_spec] * 2,
      out_specs=[_row_spec, _row_spec],
      out_shape=[jax.ShapeDtypeStruct((_N, _D), jnp.float32)] * 2,
  )(a0, a1, h0, h1, *g0, *g1)


def _gru2_transform(a0, a1, h0, h1, g0, g1, w0, b0, w1, b1):
  return pl.pallas_call(
      _gru2_tr_body,
      grid=(_N // _BN,),
      in_specs=([_row_spec] * 4 + [_w3_spec, _w3_spec, _b3_spec, _b3_spec] * 2
                + [_w_spec, _b_spec, _w_spec, _b_spec]),
      out_specs=[_row_spec, _row_spec,
                 pl.BlockSpec((_NC, _K, _BN, _D), lambda i: (0, 0, i, 0))],
      out_shape=[jax.ShapeDtypeStruct((_N, _D), jnp.float32),
                 jax.ShapeDtypeStruct((_N, _D), jnp.float32),
                 jax.ShapeDtypeStruct((_NC, _K, _N, _D), jnp.float32)],
  )(a0, a1, h0, h1, *g0, *g1, w0, b0, w1, b1)


def _post_body(h0_ref, h1_ref, g0_ref, b0_ref, g1_ref, b1_ref, out_ref):
  eps = 1e-5
  for idx, (h_ref, g_ref, b_ref) in enumerate(
      ((h0_ref, g0_ref, b0_ref), (h1_ref, g1_ref, b1_ref))):
    hv = jnp.maximum(h_ref[...], 0.0)
    mean = jnp.mean(hv, axis=0, keepdims=True)
    var = jnp.mean(jnp.square(hv - mean), axis=0, keepdims=True)
    y = g_ref[...] * (hv - mean) / jnp.sqrt(var + eps) + b_ref[...]
    out_ref[:, idx * _D:(idx + 1) * _D] = y


def _post(h0, h1, g0, b0, g1, b1):
  full = pl.BlockSpec((_N, _D), lambda: (0, 0))
  one = pl.BlockSpec((1, _D), lambda: (0, 0))
  return pl.pallas_call(
      _post_body,
      in_specs=[full, full, one, one, one, one],
      out_specs=pl.BlockSpec((_N, 2 * _D), lambda: (0, 0)),
      out_shape=jax.ShapeDtypeStruct((_N, 2 * _D), jnp.float32),
  )(h0, h1, g0.reshape(1, _D), b0.reshape(1, _D), g1.reshape(1, _D), b1.reshape(1, _D))


def _unpack(packed_v, j, base, idx_buf, dst_buf):
  for i in range(_C // 16):
    v = packed_v[pl.ds(j * _C + i * 16, 16)]
    dst_buf[pl.ds(i * 16, 16)] = lax.shift_right_logical(v, 16)
    idx_buf[pl.ds(i * 16, 16)] = lax.bitwise_and(v, 0xFFFF) + base


def _sc_scatter_body(wh_hbm, packed_hbm, zeros_hbm, out0_hbm, out1_hbm, *refs):
  packed_v = refs[0]
  rows = refs[1:1 + _NBUF]
  ibufs = refs[1 + _NBUF:1 + 2 * _NBUF]
  dbufs = refs[1 + 2 * _NBUF:1 + 3 * _NBUF]
  acc_sh = refs[1 + 3 * _NBUF]
  sem_z = refs[2 + 3 * _NBUF]
  sems = refs[3 + 3 * _NBUF:]

  cid = lax.axis_index("c")
  sid = lax.axis_index("s")
  base = cid * (_K * _N)

  zd = pltpu.async_copy(zeros_hbm.at[pl.ds(sid * _ZR, _ZR)],
                        acc_sh.at[pl.ds(sid * _ZR, _ZR)], sem_z)
  pltpu.sync_copy(packed_hbm.at[sid, pl.ds(0, _HCH * _C)], packed_v)
  zd.wait()
  plsc.subcore_barrier()

  def body(t, carry):
    c = _NBUF * t
    for b in range(_NBUF):
      pltpu.make_async_copy(wh_hbm.at[ibufs[b]], rows[b], sems[b]).wait()
      pltpu.sync_copy(rows[b], acc_sh.at[dbufs[b]], add=True)
      _unpack(packed_v, (c + _NBUF + b) % _HCH, base, ibufs[b], dbufs[b])
      pltpu.async_copy(wh_hbm.at[ibufs[b]], rows[b], sems[b])
    return carry

  for q in range(_NQ):
    if q:
      for b in range(_NBUF):
        pltpu.make_async_copy(wh_hbm.at[ibufs[b]], rows[b], sems[b]).wait()
        pltpu.sync_copy(rows[b], acc_sh.at[dbufs[b]], add=True)
      pltpu.sync_copy(packed_hbm.at[sid, pl.ds(q * _HCH * _C, _HCH * _C)], packed_v)
    for b in range(_NBUF):
      _unpack(packed_v, b, base, ibufs[b], dbufs[b])
      pltpu.async_copy(wh_hbm.at[ibufs[b]], rows[b], sems[b])
    lax.fori_loop(0, _HCH // _NBUF - 1, body, 0)
  for b in range(_NBUF):
    pltpu.make_async_copy(wh_hbm.at[ibufs[b]], rows[b], sems[b]).wait()
    pltpu.sync_copy(rows[b], acc_sh.at[dbufs[b]], add=True)
  plsc.subcore_barrier()

  @pl.when(cid == 0)
  def _():
    pltpu.sync_copy(acc_sh.at[pl.ds(sid * _ZR, _ZR)],
                    out0_hbm.at[pl.ds(sid * _ZR, _ZR)])

  @pl.when(cid == 1)
  def _():
    pltpu.sync_copy(acc_sh.at[pl.ds(sid * _ZR, _ZR)],
                    out1_hbm.at[pl.ds(sid * _ZR, _ZR)])


@functools.partial(jax.jit, static_argnames=())
def _sc_scatter(wh_flat, packed, zeros):
  mesh = plsc.VectorSubcoreMesh(core_axis_name="c", subcore_axis_name="s")
  return pl.kernel(
      _sc_scatter_body,
      out_type=[jax.ShapeDtypeStruct((_NPAD, _D), jnp.float32)] * _NC,
      mesh=mesh,
      scratch_types=(
          [pltpu.VMEM((_HCH * _C,), jnp.int32)]
          + [pltpu.VMEM((_C, _D), jnp.float32)] * _NBUF
          + [pltpu.VMEM((_C,), jnp.int32)] * (2 * _NBUF)
          + [pltpu.VMEM_SHARED((_NPAD, _D), jnp.float32)]
          + [pltpu.SemaphoreType.DMA] * (1 + _NBUF)
      ),
  )(wh_flat, packed, zeros)


def kernel(x, edge_index, etypes,
           lin_W0, lin_b0, gru_Wih0, gru_Whh0, gru_bih0, gru_bhh0, bn_g0, bn_b0,
           lin_W1, lin_b1, gru_Wih1, gru_Whh1, gru_bih1, gru_bhh1, bn_g1, bn_b1):
  src = edge_index[0].astype(jnp.int32)
  dst = edge_index[1].astype(jnp.int32)
  gidx = etypes.astype(jnp.int32) * _N + src
  ar = jnp.arange(_EPAD, dtype=jnp.int32)
  pad_gidx = (ar * 97) % (_K * _N)
  pad_dst = _N + ar % (_NPAD - _N)
  packed = (jnp.concatenate([dst, pad_dst]) << 16) | jnp.concatenate([gidx, pad_gidx])
  packed = packed.reshape(_NS, _EPW)
  zeros = jnp.zeros((_NPAD, _D), jnp.float32)

  g0 = (gru_Wih0.T, gru_Whh0.T, gru_bih0.reshape(1, -1), gru_bhh0.reshape(1, -1))
  g1 = (gru_Wih1.T, gru_Whh1.T, gru_bih1.reshape(1, -1), gru_bhh1.reshape(1, -1))

  h0 = h1 = x
  wh = _transform2(x, lin_W0, lin_b0, lin_W1, lin_b1)
  for step in range(_STEPS):
    a0, a1 = _sc_scatter(wh.reshape(_NC * _K * _N, _D), packed, zeros)
    if step < _STEPS - 1:
      h0, h1, wh = _gru2_transform(a0, a1, h0, h1, g0, g1,
                                   lin_W0, lin_b0, lin_W1, lin_b1)
    else:
      h0, h1 = _gru2(a0, a1, h0, h1, g0, g1)
  return _post(h0, h1, bn_g0, bn_b0, bn_g1, bn_b1)


# speedup vs baseline: 1.0439x; 1.0439x over previous
import functools

import jax
import jax.numpy as jnp
from jax import lax
from jax.experimental import pallas as pl
from jax.experimental.pallas import tpu as pltpu
from jax.experimental.pallas import tpu_sc as plsc

_N = 10000
_E = 320000
_D = 128
_K = 4
_STEPS = 5

_NC = 2
_NS = 16
_C = 64
_CH = 320
_NBUF = 4
_HCH = _CH // 2
_EPW = _C * _CH
_EPAD = _NS * _EPW - _E
_NPAD = 10240
_ZR = _NPAD // _NS

_BN = 2000


def _transform2_body(x_ref, w0_ref, b0_ref, w1_ref, b1_ref, out_ref):
  xv = x_ref[...]
  for l, (w_ref, b_ref) in enumerate(((w0_ref, b0_ref), (w1_ref, b1_ref))):
    for k in range(_K):
      out_ref[l, k] = (jnp.dot(xv, w_ref[k], preferred_element_type=jnp.float32)
                       + b_ref[k][None, :])


_w_spec = pl.BlockSpec((_K, _D, _D), lambda i: (0, 0, 0))
_b_spec = pl.BlockSpec((_K, _D), lambda i: (0, 0))


def _transform2(x, w0, b0, w1, b1):
  return pl.pallas_call(
      _transform2_body,
      grid=(_N // _BN,),
      in_specs=[pl.BlockSpec((_BN, _D), lambda i: (i, 0)),
                _w_spec, _b_spec, _w_spec, _b_spec],
      out_specs=pl.BlockSpec((_NC, _K, _BN, _D), lambda i: (0, 0, i, 0)),
      out_shape=jax.ShapeDtypeStruct((_NC, _K, _N, _D), jnp.float32),
  )(x, w0, b0, w1, b1)


def _gru_math(a, h, wih_t, whh_t, bih, bhh):
  gi = jnp.dot(a, wih_t, preferred_element_type=jnp.float32) + bih
  gh = jnp.dot(h, whh_t, preferred_element_type=jnp.float32) + bhh
  r = jax.nn.sigmoid(gi[:, :_D] + gh[:, :_D])
  z = jax.nn.sigmoid(gi[:, _D:2 * _D] + gh[:, _D:2 * _D])
  n = jnp.tanh(gi[:, 2 * _D:] + r * gh[:, 2 * _D:])
  return (1.0 - z) * n + z * h


def _gru2_body(a0_ref, a1_ref, h0_ref, h1_ref,
               wih0_ref, whh0_ref, bih0_ref, bhh0_ref,
               wih1_ref, whh1_ref, bih1_ref, bhh1_ref,
               h0o_ref, h1o_ref):
  h0o_ref[...] = _gru_math(a0_ref[...], h0_ref[...], wih0_ref[...],
                           whh0_ref[...], bih0_ref[...], bhh0_ref[...])
  h1o_ref[...] = _gru_math(a1_ref[...], h1_ref[...], wih1_ref[...],
                           whh1_ref[...], bih1_ref[...], bhh1_ref[...])


def _gru2_tr_body(a0_ref, a1_ref, h0_ref, h1_ref,
                  wih0_ref, whh0_ref, bih0_ref, bhh0_ref,
                  wih1_ref, whh1_ref, bih1_ref, bhh1_ref,
                  w0_ref, b0_ref, w1_ref, b1_ref,
                  h0o_ref, h1o_ref, wh_ref):
  for l, (a_ref, h_ref, wih_ref, whh_ref, bih_ref, bhh_ref, w_ref, b_ref,
          ho_ref) in enumerate((
              (a0_ref, h0_ref, wih0_ref, whh0_ref, bih0_ref, bhh0_ref,
               w0_ref, b0_ref, h0o_ref),
              (a1_ref, h1_ref, wih1_ref, whh1_ref, bih1_ref, bhh1_ref,
               w1_ref, b1_ref, h1o_ref))):
    hn = _gru_math(a_ref[...], h_ref[...], wih_ref[...], whh_ref[...],
                   bih_ref[...], bhh_ref[...])
    ho_ref[...] = hn
    for k in range(_K):
      wh_ref[l, k] = (jnp.dot(hn, w_ref[k], preferred_element_type=jnp.float32)
                      + b_ref[k][None, :])


_row_spec = pl.BlockSpec((_BN, _D), lambda i: (i, 0))
_w3_spec = pl.BlockSpec((_D, 3 * _D), lambda i: (0, 0))
_b3_spec = pl.BlockSpec((1, 3 * _D), lambda i: (0, 0))


def _gru2(a0, a1, h0, h1, g0, g1):
  return pl.pallas_call(
      _gru2_body,
      grid=(_N // _BN,),
      in_specs=[_row_spec] * 4 + [_w3_spec, _w3_spec, _b3_spec, _b3_spec] * 2,
      out_specs=[_row_spec, _row_spec],
      out_shape=[jax.ShapeDtypeStruct((_N, _D), jnp.float32)] * 2,
  )(a0, a1, h0, h1, *g0, *g1)


def _gru2_transform(a0, a1, h0, h1, g0, g1, w0, b0, w1, b1):
  return pl.pallas_call(
      _gru2_tr_body,
      grid=(_N // _BN,),
      in_specs=([_row_spec] * 4 + [_w3_spec, _w3_spec, _b3_spec, _b3_spec] * 2
                + [_w_spec, _b_spec, _w_spec, _b_spec]),
      out_specs=[_row_spec, _row_spec,
                 pl.BlockSpec((_NC, _K, _BN, _D), lambda i: (0, 0, i, 0))],
      out_shape=[jax.ShapeDtypeStruct((_N, _D), jnp.float32),
                 jax.ShapeDtypeStruct((_N, _D), jnp.float32),
                 jax.ShapeDtypeStruct((_NC, _K, _N, _D), jnp.float32)],
  )(a0, a1, h0, h1, *g0, *g1, w0, b0, w1, b1)


def _post_body(h0_ref, h1_ref, g0_ref, b0_ref, g1_ref, b1_ref, out_ref):
  eps = 1e-5
  for idx, (h_ref, g_ref, b_ref) in enumerate(
      ((h0_ref, g0_ref, b0_ref), (h1_ref, g1_ref, b1_ref))):
    hv = jnp.maximum(h_ref[...], 0.0)
    mean = jnp.mean(hv, axis=0, keepdims=True)
    var = jnp.mean(jnp.square(hv - mean), axis=0, keepdims=True)
    y = g_ref[...] * (hv - mean) / jnp.sqrt(var + eps) + b_ref[...]
    out_ref[:, idx * _D:(idx + 1) * _D] = y


def _post(h0, h1, g0, b0, g1, b1):
  full = pl.BlockSpec((_N, _D), lambda: (0, 0))
  one = pl.BlockSpec((1, _D), lambda: (0, 0))
  return pl.pallas_call(
      _post_body,
      in_specs=[full, full, one, one, one, one],
      out_specs=pl.BlockSpec((_N, 2 * _D), lambda: (0, 0)),
      out_shape=jax.ShapeDtypeStruct((_N, 2 * _D), jnp.float32),
  )(h0, h1, g0.reshape(1, _D), b0.reshape(1, _D), g1.reshape(1, _D), b1.reshape(1, _D))


def _unpack(packed_v, j, base, idx_buf, dst_buf):
  for i in range(_C // 16):
    v = packed_v[pl.ds(j * _C + i * 16, 16)]
    dst_buf[pl.ds(i * 16, 16)] = lax.shift_right_logical(v, 16)
    idx_buf[pl.ds(i * 16, 16)] = lax.bitwise_and(v, 0xFFFF) + base


def _sc_scatter_body(wh_hbm, packed_hbm, zeros_hbm, out0_hbm, out1_hbm, *refs):
  packed_v = refs[0]
  rows = refs[1:1 + _NBUF]
  ibufs = refs[1 + _NBUF:1 + 2 * _NBUF]
  dbufs = refs[1 + 2 * _NBUF:1 + 3 * _NBUF]
  acc_sh = refs[1 + 3 * _NBUF]
  sem_z = refs[2 + 3 * _NBUF]
  sems = refs[3 + 3 * _NBUF:]

  cid = lax.axis_index("c")
  sid = lax.axis_index("s")
  base = cid * (_K * _N)

  zd = pltpu.async_copy(zeros_hbm.at[pl.ds(sid * _ZR, _ZR)],
                        acc_sh.at[pl.ds(sid * _ZR, _ZR)], sem_z)
  pltpu.sync_copy(packed_hbm.at[sid, pl.ds(0, _HCH * _C)], packed_v)

  for b in range(_NBUF):
    _unpack(packed_v, b, base, ibufs[b], dbufs[b])
    pltpu.async_copy(wh_hbm.at[ibufs[b]], rows[b], sems[b])
  zd.wait()
  plsc.subcore_barrier()

  def body(t, carry):
    c = _NBUF * t
    for b in range(_NBUF):
      pltpu.make_async_copy(wh_hbm.at[ibufs[b]], rows[b], sems[b]).wait()
      pltpu.sync_copy(rows[b], acc_sh.at[dbufs[b]], add=True)
      _unpack(packed_v, (c + _NBUF + b) % _HCH, base, ibufs[b], dbufs[b])
      pltpu.async_copy(wh_hbm.at[ibufs[b]], rows[b], sems[b])
    return carry

  lax.fori_loop(0, _HCH // _NBUF - 1, body, 0)
  for b in range(_NBUF):
    pltpu.make_async_copy(wh_hbm.at[ibufs[b]], rows[b], sems[b]).wait()
    pltpu.sync_copy(rows[b], acc_sh.at[dbufs[b]], add=True)
  pltpu.sync_copy(packed_hbm.at[sid, pl.ds(_HCH * _C, _HCH * _C)], packed_v)
  for b in range(_NBUF):
    _unpack(packed_v, b, base, ibufs[b], dbufs[b])
    pltpu.async_copy(wh_hbm.at[ibufs[b]], rows[b], sems[b])
  lax.fori_loop(0, _HCH // _NBUF - 1, body, 0)
  for b in range(_NBUF):
    pltpu.make_async_copy(wh_hbm.at[ibufs[b]], rows[b], sems[b]).wait()
    pltpu.sync_copy(rows[b], acc_sh.at[dbufs[b]], add=True)
  plsc.subcore_barrier()

  @pl.when(cid == 0)
  def _():
    pltpu.sync_copy(acc_sh.at[pl.ds(sid * _ZR, _ZR)],
                    out0_hbm.at[pl.ds(sid * _ZR, _ZR)])

  @pl.when(cid == 1)
  def _():
    pltpu.sync_copy(acc_sh.at[pl.ds(sid * _ZR, _ZR)],
                    out1_hbm.at[pl.ds(sid * _ZR, _ZR)])


@functools.partial(jax.jit, static_argnames=())
def _sc_scatter(wh_flat, packed, zeros):
  mesh = plsc.VectorSubcoreMesh(core_axis_name="c", subcore_axis_name="s")
  return pl.kernel(
      _sc_scatter_body,
      out_type=[jax.ShapeDtypeStruct((_NPAD, _D), jnp.float32)] * _NC,
      mesh=mesh,
      scratch_types=(
          [pltpu.VMEM((_HCH * _C,), jnp.int32)]
          + [pltpu.VMEM((_C, _D), jnp.float32)] * _NBUF
          + [pltpu.VMEM((_C,), jnp.int32)] * (2 * _NBUF)
          + [pltpu.VMEM_SHARED((_NPAD, _D), jnp.float32)]
          + [pltpu.SemaphoreType.DMA] * (1 + _NBUF)
      ),
  )(wh_flat, packed, zeros)


def kernel(x, edge_index, etypes,
           lin_W0, lin_b0, gru_Wih0, gru_Whh0, gru_bih0, gru_bhh0, bn_g0, bn_b0,
           lin_W1, lin_b1, gru_Wih1, gru_Whh1, gru_bih1, gru_bhh1, bn_g1, bn_b1):
  src = edge_index[0].astype(jnp.int32)
  dst = edge_index[1].astype(jnp.int32)
  gidx = etypes.astype(jnp.int32) * _N + src
  ar = jnp.arange(_EPAD, dtype=jnp.int32)
  pad_gidx = (ar * 97) % (_K * _N)
  pad_dst = _N + ar % (_NPAD - _N)
  packed = (jnp.concatenate([dst, pad_dst]) << 16) | jnp.concatenate([gidx, pad_gidx])
  packed = packed.reshape(_NS, _EPW)
  zeros = jnp.zeros((_NPAD, _D), jnp.float32)

  g0 = (gru_Wih0.T, gru_Whh0.T, gru_bih0.reshape(1, -1), gru_bhh0.reshape(1, -1))
  g1 = (gru_Wih1.T, gru_Whh1.T, gru_bih1.reshape(1, -1), gru_bhh1.reshape(1, -1))

  h0 = h1 = x
  wh = _transform2(x, lin_W0, lin_b0, lin_W1, lin_b1)
  for step in range(_STEPS):
    a0, a1 = _sc_scatter(wh.reshape(_NC * _K * _N, _D), packed, zeros)
    if step < _STEPS - 1:
      h0, h1, wh = _gru2_transform(a0, a1, h0, h1, g0, g1,
                                   lin_W0, lin_b0, lin_W1, lin_b1)
    else:
      h0, h1 = _gru2(a0, a1, h0, h1, g0, g1)
  return _post(h0, h1, bn_g0, bn_b0, bn_g1, bn_b1)
